# Initial kernel scaffold; baseline (speedup 1.0000x reference)
import jax
import jax.numpy as jnp
from jax.experimental import pallas as pl


def kernel(pos_indicator, pred_confs, rand_vals):
    raise NotImplementedError("write your pallas kernel here")


# speedup vs baseline: 3.9550x; 3.9550x over previous
import functools

import jax
import jax.numpy as jnp
from jax import lax
from jax.experimental import pallas as pl
from jax.experimental.pallas import tpu as pltpu
from jax.experimental.pallas import tpu_sc as plsc

B = 8
HW = 160 * 160
L = 16
NSLICES = HW // L
K = 512
EPS = 1e-7
MAXBITS = 0x3F7FFFFF


def _count_ge(ref, thresh, unroll=16):
    tv = jnp.full((L,), thresh, jnp.int32)
    zero = jnp.zeros((L,), jnp.int32)
    one = jnp.ones((L,), jnp.int32)

    def body(i, accs):
        accs = list(accs)
        base = i * (unroll * L)
        for j in range(unroll):
            v = ref[pl.ds(base + j * L, L)]
            accs[j % 4] = accs[j % 4] + jnp.where(v >= tv, one, zero)
        return tuple(accs)

    accs = lax.fori_loop(0, NSLICES // unroll, body, (zero, zero, zero, zero))
    return jnp.sum(accs[0] + accs[1] + accs[2] + accs[3])


def _search(ref, k):
    def step(_, lohi):
        lo, hi = lohi
        mid = lax.shift_right_logical(lo + hi + 1, 1)
        big = _count_ge(ref, mid) >= k
        return (jnp.where(big, mid, lo), jnp.where(big, hi, mid - 1))

    lo, _ = lax.fori_loop(0, 30, step, (jnp.int32(0), jnp.int32(MAXBITS)))
    return lo


def _fots_body(pos_hbm, conf_hbm, cb_hbm, rvb_hbm, out_hbm,
               a_ref, c_ref, d_ref, r_ref, o_ref):
    wid = lax.axis_index("s") * 2 + lax.axis_index("c")

    @pl.when(wid < B)
    def _():
        pltpu.sync_copy(pos_hbm.at[wid], a_ref)
        pltpu.sync_copy(conf_hbm.at[wid], c_ref)
        pltpu.sync_copy(cb_hbm.at[wid], d_ref)
        pltpu.sync_copy(rvb_hbm.at[wid], r_ref)

        zi = jnp.zeros((L,), jnp.int32)
        zf = jnp.zeros((L,), jnp.float32)
        neg1 = jnp.full((L,), -1, jnp.int32)

        def p0(i, carry):
            np_acc, ps_acc = carry
            for j in range(8):
                sl = pl.ds(i * (8 * L) + j * L, L)
                p = a_ref[sl]
                cv = c_ref[sl]
                pm = p > zi
                np_acc = np_acc + p
                ps_acc = ps_acc + jnp.where(pm, cv, zf)
                d_ref[sl] = jnp.where(pm, neg1, d_ref[sl])
            return np_acc, ps_acc

        np_acc, ps_acc = lax.fori_loop(0, NSLICES // 8, p0, (zi, zf))
        n_pos = jnp.sum(np_acc).astype(jnp.float32)
        pos_sum = jnp.sum(ps_acc)

        t1 = _search(d_ref, K)
        t1v = jnp.full((L,), t1, jnp.int32)

        def p1(i, hs_acc):
            for j in range(8):
                sl = pl.ds(i * (8 * L) + j * L, L)
                mcb = d_ref[sl]
                cv = c_ref[sl]
                rb = r_ref[sl]
                hs_acc = hs_acc + jnp.where(mcb >= t1v, cv, zf)
                rem = (mcb >= zi) & (mcb < t1v)
                r_ref[sl] = jnp.where(rem, rb, neg1)
            return hs_acc

        hard_sum = jnp.sum(lax.fori_loop(0, NSLICES // 8, p1, zf))

        t2 = _search(r_ref, K)
        t2v = jnp.full((L,), t2, jnp.int32)

        def p2(i, rs_acc):
            for j in range(8):
                sl = pl.ds(i * (8 * L) + j * L, L)
                mrv = r_ref[sl]
                cv = c_ref[sl]
                rs_acc = rs_acc + jnp.where(mrv >= t2v, cv, zf)
            return rs_acc

        rand_sum = jnp.sum(lax.fori_loop(0, NSLICES // 8, p2, zf))

        eps = jnp.full((L,), EPS, jnp.float32)
        one = jnp.ones((L,), jnp.float32)
        two = jnp.full((L,), 2.0, jnp.float32)
        ps_v = jnp.full((L,), pos_sum, jnp.float32)
        np_v = jnp.full((L,), n_pos, jnp.float32)
        hs_v = jnp.full((L,), hard_sum, jnp.float32)
        rs_v = jnp.full((L,), rand_sum, jnp.float32)
        pos_loss = one - two * (ps_v + eps) / (ps_v + np_v + eps)
        hard_loss = one - two * eps / (hs_v + eps)
        rand_loss = one - two * eps / (rs_v + eps)
        loss = (pos_loss + hard_loss + rand_loss) / jnp.full((L,), 2 * K, jnp.float32)

        o_ref[...] = loss
        pltpu.sync_copy(o_ref, out_hbm.at[wid])


_fots = functools.partial(
    pl.kernel,
    out_type=jax.ShapeDtypeStruct((B, L), jnp.float32),
    mesh=plsc.VectorSubcoreMesh(core_axis_name="c", subcore_axis_name="s"),
    scratch_types=[
        pltpu.VMEM((HW,), jnp.int32),
        pltpu.VMEM((HW,), jnp.float32),
        pltpu.VMEM((HW,), jnp.int32),
        pltpu.VMEM((HW,), jnp.int32),
        pltpu.VMEM((L,), jnp.float32),
    ],
    compiler_params=pltpu.CompilerParams(needs_layout_passes=False),
)(_fots_body)


def kernel(pos_indicator, pred_confs, rand_vals):
    pos_i = pos_indicator.reshape(B, HW).astype(jnp.int32)
    conf = pred_confs.reshape(B, HW)
    cb = lax.bitcast_convert_type(conf, jnp.int32)
    rvb = lax.bitcast_convert_type(rand_vals, jnp.int32)
    out = _fots(pos_i, conf, cb, rvb)
    return out[:, 0]
